# baseline (speedup 1.0000x reference)
import functools

import jax
import jax.numpy as jnp
from jax import lax
from jax.experimental import pallas as pl
from jax.experimental.pallas import tpu as pltpu
from jax.experimental.pallas import tpu_sc as plsc

N = 10000
E = 320000
G = 16
DH = 64

NC, NS = 2, 16
NW = NC * NS
SUB = 128
CHUNK = 640
NSUB = CHUNK // SUB
EPT = 10240
NCHUNK = EPT // CHUNK
NSUBT = EPT // SUB
NBUF = 8
LOOKAHEAD = 5
EPAD = EPT * NW
NPAD = 10240
RPT = NPAD // NS

_MESH = plsc.VectorSubcoreMesh(core_axis_name="c", subcore_axis_name="s")
_SC_PARAMS = pltpu.CompilerParams(use_tc_tiling_on_sc=False)
_F32 = jnp.float32


def _zero_vmem(ref, rows, cols):
    z = jnp.zeros((16,), _F32)
    for r in range(rows):
        for k in range(cols // 16):
            ref[r, pl.ds(k * 16, 16)] = z


@functools.partial(
    pl.kernel,
    out_type=jax.ShapeDtypeStruct((NC, NPAD), _F32),
    mesh=_MESH,
    compiler_params=_SC_PARAMS,
    scratch_types=[
        pltpu.VMEM((NSUB, SUB), jnp.int32),
        pltpu.VMEM((SUB,), _F32),
        pltpu.VMEM((RPT,), _F32),
        pltpu.VMEM_SHARED((NPAD,), _F32),
        pltpu.SemaphoreType.DMA,
    ],
)
def _sc_degree(dst_hbm, out_hbm, idx_v, ones_v, zrow_v, acc_sh, sem):
    c = lax.axis_index("c")
    s = lax.axis_index("s")
    wid = c * NS + s
    for k in range(SUB // 16):
        ones_v[pl.ds(k * 16, 16)] = jnp.ones((16,), _F32)
    for k in range(RPT // 16):
        zrow_v[pl.ds(k * 16, 16)] = jnp.zeros((16,), _F32)
    pltpu.sync_copy(zrow_v, acc_sh.at[pl.ds(s * RPT, RPT)])
    plsc.subcore_barrier()
    for i in range(NCHUNK):
        rowbase = wid * (EPT // SUB) + i * NSUB
        pltpu.sync_copy(dst_hbm.at[pl.ds(rowbase, NSUB)], idx_v)
        cps = [
            pltpu.async_copy(ones_v, acc_sh.at[idx_v.at[j]], sem, add=True)
            for j in range(NSUB)
        ]
        for cp in cps:
            cp.wait()
    plsc.subcore_barrier()
    pltpu.sync_copy(acc_sh.at[pl.ds(s * RPT, RPT)], out_hbm.at[c, pl.ds(s * RPT, RPT)])


@functools.partial(
    pl.kernel,
    out_type=jax.ShapeDtypeStruct((NC, NPAD, DH), _F32),
    mesh=_MESH,
    compiler_params=_SC_PARAMS,
    scratch_types=[
        pltpu.VMEM((NSUBT, SUB), jnp.int32),
        pltpu.VMEM((NSUBT, SUB), jnp.int32),
        pltpu.VMEM((NBUF, SUB, DH), _F32),
        pltpu.VMEM((32, DH), _F32),
        pltpu.VMEM_SHARED((NPAD, DH), _F32),
    ] + [pltpu.SemaphoreType.DMA] * (2 * NBUF),
)
def _sc_agg(g_hbm, src_hbm, dst_hbm, out_hbm,
            idx_s, idx_d, rows_v, zb_v, acc_sh, *sems):
    c = lax.axis_index("c")
    s = lax.axis_index("s")
    wid = c * NS + s
    sem_g = sems[:NBUF]
    sem_s = sems[NBUF:]
    rowbase = wid * NSUBT
    pltpu.sync_copy(src_hbm.at[pl.ds(rowbase, NSUBT)], idx_s)
    pltpu.sync_copy(dst_hbm.at[pl.ds(rowbase, NSUBT)], idx_d)
    _zero_vmem(zb_v, 32, DH)
    for j in range(RPT // 32):
        pltpu.sync_copy(zb_v, acc_sh.at[pl.ds(s * RPT + j * 32, 32)])
    plsc.subcore_barrier()

    gets = [None] * NBUF
    puts = [None] * NBUF
    next_g = 0
    for i in range(NSUBT):
        while next_g <= min(i + LOOKAHEAD, NSUBT - 1):
            b = next_g % NBUF
            if puts[b] is not None:
                puts[b].wait()
                puts[b] = None
            gets[b] = pltpu.async_copy(
                g_hbm.at[idx_s.at[next_g]], rows_v.at[b], sem_g[b])
            next_g += 1
        b = i % NBUF
        gets[b].wait()
        puts[b] = pltpu.async_copy(
            rows_v.at[b], acc_sh.at[idx_d.at[i]], sem_s[b], add=True)
    for b in range(NBUF):
        if puts[b] is not None:
            puts[b].wait()
    plsc.subcore_barrier()
    pltpu.sync_copy(acc_sh.at[pl.ds(s * RPT, RPT)],
                    out_hbm.at[c, pl.ds(s * RPT, RPT)])


def _tc_call(body, out_shape, *args):
    return pl.pallas_call(body, out_shape=out_shape)(*args)


def _mm0_body(x_ref, w_ref, u_ref):
    u_ref[...] = jnp.dot(x_ref[...], w_ref[...], preferred_element_type=_F32)


def _dinv_scale_body(u_ref, degp_ref, g_ref, dinv_ref):
    deg = degp_ref[0, :N] + degp_ref[1, :N] + 1.0
    dinv = lax.rsqrt(deg)[:, None]
    dinv_ref[...] = dinv
    g_ref[...] = u_ref[...] * dinv


def _layer_body(p_ref, g_ref, dinv_ref, b_ref, w_ref, o_ref):
    psum = p_ref[0, :N] + p_ref[1, :N]
    h = dinv_ref[...] * (psum + g_ref[...]) + b_ref[...]
    h = jnp.maximum(h, 0.0)
    o_ref[...] = jnp.dot(h, w_ref[...], preferred_element_type=_F32) * dinv_ref[...]


def _final_body(p_ref, g_ref, dinv_ref, b_ref, batch_ref,
                wo1_ref, bo1_ref, wo2_ref, bo2_ref, o_ref):
    psum = p_ref[0, :N] + p_ref[1, :N]
    h = dinv_ref[...] * (psum + g_ref[...]) + b_ref[...]
    h = jnp.maximum(h, 0.0)
    gid = lax.broadcasted_iota(jnp.int32, (N, G), 1)
    onehot = jnp.where(batch_ref[...] == gid, 1.0, 0.0)
    sums = lax.dot_general(onehot, h, (((0,), (0,)), ((), ())),
                           preferred_element_type=_F32)
    counts = jnp.sum(onehot, axis=0)[:, None]
    pooled = sums / jnp.maximum(counts, 1.0)
    hid = jnp.maximum(
        jnp.dot(pooled, wo1_ref[...], preferred_element_type=_F32)
        + bo1_ref[...], 0.0)
    o_ref[...] = jnp.dot(hid, wo2_ref[...],
                         preferred_element_type=_F32) + bo2_ref[...]


def kernel(x, edge_index, batch, W0, b0, W1, b1, W2, b2, W3, b3,
           Wo1, bo1, Wo2, bo2):
    npad = EPAD - E
    pad_src = jnp.arange(npad, dtype=jnp.int32) % N
    pad_dst = N + jnp.arange(npad, dtype=jnp.int32) % (NPAD - N)
    src = jnp.concatenate([edge_index[0], pad_src]).reshape(EPAD // SUB, SUB)
    dst = jnp.concatenate([edge_index[1], pad_dst]).reshape(EPAD // SUB, SUB)

    u = _tc_call(_mm0_body, jax.ShapeDtypeStruct((N, DH), _F32), x, W0)
    degp = _sc_degree(dst)

    g, dinv = pl.pallas_call(
        _dinv_scale_body,
        out_shape=(jax.ShapeDtypeStruct((N, DH), _F32),
                   jax.ShapeDtypeStruct((N, 1), _F32)),
    )(u, degp)

    for (b, Wn) in ((b0, W1), (b1, W2), (b2, W3)):
        p = _sc_agg(g, src, dst)
        g = _tc_call(_layer_body, jax.ShapeDtypeStruct((N, DH), _F32),
                     p, g, dinv, b[None, :], Wn)

    p = _sc_agg(g, src, dst)
    out = _tc_call(_final_body, jax.ShapeDtypeStruct((G, 2), _F32),
                   p, g, dinv, b3[None, :],
                   batch[:, None], Wo1, bo1[None, :], Wo2, bo2[None, :])
    return out


# speedup vs baseline: 1.0045x; 1.0045x over previous
import functools

import jax
import jax.numpy as jnp
from jax import lax
from jax.experimental import pallas as pl
from jax.experimental.pallas import tpu as pltpu
from jax.experimental.pallas import tpu_sc as plsc

N = 10000
E = 320000
G = 16
DH = 64

NC, NS = 2, 16
NW = NC * NS
SUB = 128
CHUNK = 640
NSUB = CHUNK // SUB
EPT = 10240
NCHUNK = EPT // CHUNK
NSUBT = EPT // SUB
NBUF = 8
LOOKAHEAD = 5
EPAD = EPT * NW
NPAD = 10240
RPT = NPAD // NS

_MESH = plsc.VectorSubcoreMesh(core_axis_name="c", subcore_axis_name="s")
_SC_PARAMS = pltpu.CompilerParams(use_tc_tiling_on_sc=False)
_F32 = jnp.float32


def _zero_vmem(ref, rows, cols):
    z = jnp.zeros((16,), _F32)
    for r in range(rows):
        for k in range(cols // 16):
            ref[r, pl.ds(k * 16, 16)] = z


@functools.partial(
    pl.kernel,
    out_type=jax.ShapeDtypeStruct((NC, NPAD), _F32),
    mesh=_MESH,
    compiler_params=_SC_PARAMS,
    scratch_types=[
        pltpu.VMEM((NSUB, SUB), jnp.int32),
        pltpu.VMEM((SUB,), _F32),
        pltpu.VMEM((RPT,), _F32),
        pltpu.VMEM_SHARED((NPAD,), _F32),
        pltpu.SemaphoreType.DMA,
    ],
)
def _sc_degree(dst_hbm, out_hbm, idx_v, ones_v, zrow_v, acc_sh, sem):
    c = lax.axis_index("c")
    s = lax.axis_index("s")
    wid = c * NS + s
    for k in range(SUB // 16):
        ones_v[pl.ds(k * 16, 16)] = jnp.ones((16,), _F32)
    for k in range(RPT // 16):
        zrow_v[pl.ds(k * 16, 16)] = jnp.zeros((16,), _F32)
    pltpu.sync_copy(zrow_v, acc_sh.at[pl.ds(s * RPT, RPT)])
    plsc.subcore_barrier()
    for i in range(NCHUNK):
        rowbase = wid * (EPT // SUB) + i * NSUB
        pltpu.sync_copy(dst_hbm.at[pl.ds(rowbase, NSUB)], idx_v)
        cps = [
            pltpu.async_copy(ones_v, acc_sh.at[idx_v.at[j]], sem, add=True)
            for j in range(NSUB)
        ]
        for cp in cps:
            cp.wait()
    plsc.subcore_barrier()
    pltpu.sync_copy(acc_sh.at[pl.ds(s * RPT, RPT)], out_hbm.at[c, pl.ds(s * RPT, RPT)])


@functools.partial(
    pl.kernel,
    out_type=jax.ShapeDtypeStruct((NC, NPAD, DH), _F32),
    mesh=_MESH,
    compiler_params=_SC_PARAMS,
    scratch_types=[
        pltpu.VMEM((NSUBT, SUB), jnp.int32),
        pltpu.VMEM((NSUBT, SUB), jnp.int32),
        pltpu.VMEM((NBUF, SUB, DH), _F32),
        pltpu.VMEM((32, DH), _F32),
        pltpu.VMEM_SHARED((NPAD, DH), _F32),
    ] + [pltpu.SemaphoreType.DMA] * (2 * NBUF),
)
def _sc_agg(g_hbm, src_hbm, dst_hbm, out_hbm,
            idx_s, idx_d, rows_v, zb_v, acc_sh, *sems):
    c = lax.axis_index("c")
    s = lax.axis_index("s")
    wid = c * NS + s
    sem_g = sems[:NBUF]
    sem_s = sems[NBUF:]
    rowbase = wid * NSUBT
    pltpu.sync_copy(src_hbm.at[pl.ds(rowbase, NSUBT)], idx_s)
    pltpu.sync_copy(dst_hbm.at[pl.ds(rowbase, NSUBT)], idx_d)
    _zero_vmem(zb_v, 32, DH)
    for j in range(RPT // 32):
        pltpu.sync_copy(zb_v, acc_sh.at[pl.ds(s * RPT + j * 32, 32)])
    plsc.subcore_barrier()

    gets = [None] * NBUF
    puts = [None] * NBUF
    next_g = 0
    for i in range(NSUBT):
        while next_g <= min(i + LOOKAHEAD, NSUBT - 1):
            b = next_g % NBUF
            if puts[b] is not None:
                puts[b].wait()
                puts[b] = None
            gets[b] = pltpu.async_copy(
                g_hbm.at[idx_s.at[next_g]], rows_v.at[b], sem_g[b])
            next_g += 1
        b = i % NBUF
        gets[b].wait()
        puts[b] = pltpu.async_copy(
            rows_v.at[b], acc_sh.at[idx_d.at[i]], sem_s[b], add=True)
    for b in range(NBUF):
        if puts[b] is not None:
            puts[b].wait()
    plsc.subcore_barrier()
    pltpu.sync_copy(acc_sh.at[pl.ds(s * RPT, RPT)],
                    out_hbm.at[c, pl.ds(s * RPT, RPT)])


def _tc_call(body, out_shape, *args):
    return pl.pallas_call(body, out_shape=out_shape)(*args)


def _mm0_dinv_body(x_ref, w_ref, degp_ref, g_ref, dinv_ref):
    u = jnp.dot(x_ref[...], w_ref[...], preferred_element_type=_F32)
    deg = degp_ref[0, :N] + degp_ref[1, :N] + 1.0
    dinv = lax.rsqrt(deg)[:, None]
    dinv_ref[...] = dinv
    g_ref[...] = u * dinv


def _layer_body(p_ref, g_ref, dinv_ref, b_ref, w_ref, o_ref):
    psum = p_ref[0, :N] + p_ref[1, :N]
    h = dinv_ref[...] * (psum + g_ref[...]) + b_ref[...]
    h = jnp.maximum(h, 0.0)
    o_ref[...] = jnp.dot(h, w_ref[...], preferred_element_type=_F32) * dinv_ref[...]


def _final_body(p_ref, g_ref, dinv_ref, b_ref, batch_ref,
                wo1_ref, bo1_ref, wo2_ref, bo2_ref, o_ref):
    psum = p_ref[0, :N] + p_ref[1, :N]
    h = dinv_ref[...] * (psum + g_ref[...]) + b_ref[...]
    h = jnp.maximum(h, 0.0)
    gid = lax.broadcasted_iota(jnp.int32, (N, G), 1)
    onehot = jnp.where(batch_ref[...] == gid, 1.0, 0.0)
    sums = lax.dot_general(onehot, h, (((0,), (0,)), ((), ())),
                           preferred_element_type=_F32)
    counts = jnp.sum(onehot, axis=0)[:, None]
    pooled = sums / jnp.maximum(counts, 1.0)
    hid = jnp.maximum(
        jnp.dot(pooled, wo1_ref[...], preferred_element_type=_F32)
        + bo1_ref[...], 0.0)
    o_ref[...] = jnp.dot(hid, wo2_ref[...],
                         preferred_element_type=_F32) + bo2_ref[...]


def kernel(x, edge_index, batch, W0, b0, W1, b1, W2, b2, W3, b3,
           Wo1, bo1, Wo2, bo2):
    npad = EPAD - E
    pad_src = jnp.arange(npad, dtype=jnp.int32) % N
    pad_dst = N + jnp.arange(npad, dtype=jnp.int32) % (NPAD - N)
    src = jnp.concatenate([edge_index[0], pad_src]).reshape(EPAD // SUB, SUB)
    dst = jnp.concatenate([edge_index[1], pad_dst]).reshape(EPAD // SUB, SUB)

    degp = _sc_degree(dst)

    g, dinv = pl.pallas_call(
        _mm0_dinv_body,
        out_shape=(jax.ShapeDtypeStruct((N, DH), _F32),
                   jax.ShapeDtypeStruct((N, 1), _F32)),
    )(x, W0, degp)

    for (b, Wn) in ((b0, W1), (b1, W2), (b2, W3)):
        p = _sc_agg(g, src, dst)
        g = _tc_call(_layer_body, jax.ShapeDtypeStruct((N, DH), _F32),
                     p, g, dinv, b[None, :], Wn)

    p = _sc_agg(g, src, dst)
    out = _tc_call(_final_body, jax.ShapeDtypeStruct((G, 2), _F32),
                   p, g, dinv, b3[None, :],
                   batch[:, None], Wo1, bo1[None, :], Wo2, bo2[None, :])
    return out
